# baseline (speedup 1.0000x reference)
import functools

import jax
import jax.numpy as jnp
from jax.experimental import pallas as pl
from jax.experimental.pallas import tpu as pltpu

_PAD = 8


def _linear_kernel(z_ref, w_ref, b_ref, o_ref):
    acc = jnp.dot(z_ref[...], w_ref[...], preferred_element_type=jnp.float32)
    o_ref[...] = (acc + b_ref[...]).astype(o_ref.dtype)


def _linear(z, w, b, *, tn=8192):
    B, K = z.shape
    N = w.shape[1]
    tn = min(tn, N)
    return pl.pallas_call(
        _linear_kernel,
        out_shape=jax.ShapeDtypeStruct((B, N), jnp.bfloat16),
        grid=(N // tn,),
        in_specs=[
            pl.BlockSpec((B, K), lambda j: (0, 0)),
            pl.BlockSpec((K, tn), lambda j: (0, j)),
            pl.BlockSpec((1, tn), lambda j: (0, j)),
        ],
        out_specs=pl.BlockSpec((B, tn), lambda j: (0, j)),
        compiler_params=pltpu.CompilerParams(dimension_semantics=("parallel",)),
    )(z.astype(jnp.bfloat16), w.astype(jnp.bfloat16), b.reshape(1, N))


def _pconv_kernel(x_ref, s_ref, t_ref, w_ref, b_ref, *rest,
                  BI, h, w, up, pre_act, slope, act, stats, taps, P_in):
    if stats:
        o_ref, sum_ref, sq_ref, pad_ref = rest
    else:
        o_ref, pad_ref = rest
    pad_ref[...] = jnp.zeros_like(pad_ref)
    tsum = tsq = None
    for bi in range(BI):
        for pin in range(P_in):
            v = x_ref[bi, pin].astype(jnp.float32)
            if pre_act:
                v = v * s_ref[...] + t_ref[...]
                v = jnp.where(v >= 0.0, v, slope * v)
            pad_ref[pin, 1:h + 1, _PAD:_PAD + w, :] = v.astype(jnp.bfloat16)

        for k, tap_list in enumerate(taps):
            if up:
                ys = []
                for a in range(2):
                    acc = None
                    for (pin, dc, cc) in tap_list:
                        for u in range(2):
                            patch = pad_ref[pin, a + u:a + u + h,
                                            _PAD + dc:_PAD + dc + w, :]
                            d = jax.lax.dot_general(
                                patch, w_ref[(2 * a + u) * 4 + cc],
                                dimension_numbers=(((2,), (0,)), ((), ())),
                                preferred_element_type=jnp.float32,
                            )
                            acc = d if acc is None else acc + d
                    ys.append(acc + b_ref[...])
                cout = ys[0].shape[-1]
                y = jnp.stack(ys, axis=1).reshape(2 * h, w, cout)
            else:
                acc = None
                for (pin, dh, dc, t) in tap_list:
                    patch = pad_ref[pin, dh:dh + h,
                                    _PAD + dc:_PAD + dc + w, :]
                    d = jax.lax.dot_general(
                        patch, w_ref[t],
                        dimension_numbers=(((2,), (0,)), ((), ())),
                        preferred_element_type=jnp.float32,
                    )
                    acc = d if acc is None else acc + d
                y = acc + b_ref[...]
            if act == "tanh":
                y = jnp.tanh(y)
            o_ref[bi, k] = y.astype(o_ref.dtype)
            if stats:
                s = jnp.sum(jnp.sum(y, axis=0), axis=0, keepdims=True)
                q = jnp.sum(jnp.sum(y * y, axis=0), axis=0, keepdims=True)
                tsum = s if tsum is None else tsum + s
                tsq = q if tsq is None else tsq + q
    if stats:
        sum_ref[0] = tsum
        sq_ref[0] = tsq


def _pconv(x, wt, b, *, taps, P_out, up, scale=None, shift=None, slope=0.2,
           act="none", out_dtype=jnp.bfloat16, stats=True, bi=4):
    B, P_in, h, w, Cin = x.shape
    H = 2 * h if up else h
    T = wt.shape[0]
    Cout = wt.shape[-1]
    bi = max(1, min(bi, B))
    pre_act = scale is not None
    if scale is None:
        scale = jnp.ones((Cin,), jnp.float32)
        shift = jnp.zeros((Cin,), jnp.float32)
    kern = functools.partial(_pconv_kernel, BI=bi, h=h, w=w, up=up,
                             pre_act=pre_act, slope=slope, act=act,
                             stats=stats, taps=taps, P_in=P_in)
    out_shape = [jax.ShapeDtypeStruct((B, P_out, H, w, Cout), out_dtype)]
    out_specs = [pl.BlockSpec((bi, P_out, H, w, Cout),
                              lambda i: (i, 0, 0, 0, 0))]
    if stats:
        out_shape += [jax.ShapeDtypeStruct((B // bi, 1, Cout), jnp.float32)] * 2
        out_specs += [pl.BlockSpec((1, 1, Cout), lambda i: (i, 0, 0))] * 2
    res = pl.pallas_call(
        kern,
        out_shape=tuple(out_shape),
        grid_spec=pltpu.PrefetchScalarGridSpec(
            num_scalar_prefetch=0,
            grid=(B // bi,),
            in_specs=[
                pl.BlockSpec((bi, P_in, h, w, Cin), lambda i: (i, 0, 0, 0, 0)),
                pl.BlockSpec((1, Cin), lambda i: (0, 0)),
                pl.BlockSpec((1, Cin), lambda i: (0, 0)),
                pl.BlockSpec((T, Cin, Cout), lambda i: (0, 0, 0)),
                pl.BlockSpec((1, Cout), lambda i: (0, 0)),
            ],
            out_specs=out_specs,
            scratch_shapes=[
                pltpu.VMEM((P_in, h + 2, w + 2 * _PAD, Cin), jnp.bfloat16)],
        ),
        compiler_params=pltpu.CompilerParams(
            dimension_semantics=("parallel",),
            vmem_limit_bytes=100 * 1024 * 1024,
        ),
    )(x, scale.reshape(1, Cin), shift.reshape(1, Cin),
      wt.astype(jnp.bfloat16), b.reshape(1, Cout))
    return res if stats else res[0]


def _comb_weights(w):
    m = jnp.array([[1, 0, 0], [0, 1, 1], [1, 1, 0], [0, 0, 1]], w.dtype)
    wc = jnp.einsum('id,jw,dwco->ijco', m, m, w)
    return wc.reshape(16, w.shape[2], w.shape[3])


_T1 = [
    [(0, -1, 0), (0, 0, 1)],
    [(0, 0, 2), (0, 1, 3)],
]
_T2 = [
    [(1, -1, 0), (0, 0, 1)],
    [(0, 0, 2), (1, 0, 3)],
    [(0, 0, 0), (1, 0, 1)],
    [(1, 0, 2), (0, 1, 3)],
]
_T3 = [
    [((p + dw - 1) % 4, dh, (p + dw - 1) // 4, 3 * dh + dw)
     for dw in range(3) for dh in range(3)]
    for p in range(4)
]


def _bn_scale_shift(ssum, ssq, gamma, beta, count, eps=0.8):
    mean = jnp.sum(ssum, axis=(0, 1)) / count
    var = jnp.sum(ssq, axis=(0, 1)) / count - mean * mean
    scale = gamma * jax.lax.rsqrt(var + eps)
    shift = beta - mean * scale
    return scale, shift


def kernel(z, l1_w, l1_b, c1_w, c1_b, bn1_g, bn1_b,
           c2_w, c2_b, bn2_g, bn2_b, c3_w, c3_b):
    B = z.shape[0]
    init = 16

    h = _linear(z, l1_w, l1_b)
    x = h.reshape(B, 1, init, init, 128)

    c1, s1, q1 = _pconv(x, _comb_weights(c1_w), c1_b, taps=_T1, P_out=2,
                        up=True)
    sc1, sh1 = _bn_scale_shift(s1, q1, bn1_g, bn1_b, B * 32 * 32)

    c2, s2, q2 = _pconv(c1, _comb_weights(c2_w), c2_b, taps=_T2, P_out=4,
                        up=True, scale=sc1, shift=sh1)
    sc2, sh2 = _bn_scale_shift(s2, q2, bn2_g, bn2_b, B * 64 * 64)

    c3 = _pconv(c2, c3_w.reshape(9, 64, 3), c3_b, taps=_T3, P_out=4,
                up=False, scale=sc2, shift=sh2, act="tanh",
                out_dtype=jnp.float32, stats=False)

    out = c3.transpose(0, 4, 2, 3, 1).reshape(B, 3, 64, 64)
    return out


# speedup vs baseline: 1.0031x; 1.0031x over previous
import functools

import jax
import jax.numpy as jnp
from jax.experimental import pallas as pl
from jax.experimental.pallas import tpu as pltpu

_PAD = 8


def _linear_kernel(z_ref, w_ref, b_ref, o_ref):
    acc = jnp.dot(z_ref[...], w_ref[...], preferred_element_type=jnp.float32)
    o_ref[...] = (acc + b_ref[...]).astype(o_ref.dtype)


def _linear(z, w, b, *, tn=8192):
    B, K = z.shape
    N = w.shape[1]
    tn = min(tn, N)
    return pl.pallas_call(
        _linear_kernel,
        out_shape=jax.ShapeDtypeStruct((B, N), jnp.bfloat16),
        grid=(N // tn,),
        in_specs=[
            pl.BlockSpec((B, K), lambda j: (0, 0)),
            pl.BlockSpec((K, tn), lambda j: (0, j)),
            pl.BlockSpec((1, tn), lambda j: (0, j)),
        ],
        out_specs=pl.BlockSpec((B, tn), lambda j: (0, j)),
        compiler_params=pltpu.CompilerParams(dimension_semantics=("parallel",)),
    )(z.astype(jnp.bfloat16), w.astype(jnp.bfloat16), b.reshape(1, N))


def _pconv_kernel(x_ref, s_ref, t_ref, w_ref, b_ref, *rest,
                  BI, h, w, up, pre_act, slope, act, stats, taps, P_in):
    if stats:
        o_ref, sum_ref, sq_ref, pad_ref = rest
    else:
        o_ref, pad_ref = rest
    pad_ref[...] = jnp.zeros_like(pad_ref)
    tsum = tsq = None
    for bi in range(BI):
        for pin in range(P_in):
            v = x_ref[bi, pin].astype(jnp.float32)
            if pre_act:
                v = v * s_ref[...] + t_ref[...]
                v = jnp.where(v >= 0.0, v, slope * v)
            pad_ref[pin, 1:h + 1, _PAD:_PAD + w, :] = v.astype(jnp.bfloat16)

        for k, tap_list in enumerate(taps):
            if up:
                ys = []
                for a in range(2):
                    acc = None
                    for (pin, dc, cc) in tap_list:
                        for u in range(2):
                            patch = pad_ref[pin, a + u:a + u + h,
                                            _PAD + dc:_PAD + dc + w, :]
                            d = jax.lax.dot_general(
                                patch, w_ref[(2 * a + u) * 4 + cc],
                                dimension_numbers=(((2,), (0,)), ((), ())),
                                preferred_element_type=jnp.float32,
                            )
                            acc = d if acc is None else acc + d
                    ys.append(acc + b_ref[...])
                cout = ys[0].shape[-1]
                y = jnp.stack(ys, axis=1).reshape(2 * h, w, cout)
            else:
                acc = None
                for (pin, dh, dc, t) in tap_list:
                    patch = pad_ref[pin, dh:dh + h,
                                    _PAD + dc:_PAD + dc + w, :]
                    d = jax.lax.dot_general(
                        patch, w_ref[t],
                        dimension_numbers=(((2,), (0,)), ((), ())),
                        preferred_element_type=jnp.float32,
                    )
                    acc = d if acc is None else acc + d
                y = acc + b_ref[...]
            if act == "tanh":
                y = jnp.tanh(y)
            o_ref[bi, k] = y.astype(o_ref.dtype)
            if stats:
                s = jnp.sum(jnp.sum(y, axis=0), axis=0, keepdims=True)
                q = jnp.sum(jnp.sum(y * y, axis=0), axis=0, keepdims=True)
                tsum = s if tsum is None else tsum + s
                tsq = q if tsq is None else tsq + q
    if stats:
        sum_ref[0] = tsum
        sq_ref[0] = tsq


def _pconv(x, wt, b, *, taps, P_out, up, scale=None, shift=None, slope=0.2,
           act="none", out_dtype=jnp.bfloat16, stats=True, bi=8):
    B, P_in, h, w, Cin = x.shape
    H = 2 * h if up else h
    T = wt.shape[0]
    Cout = wt.shape[-1]
    bi = max(1, min(bi, B))
    pre_act = scale is not None
    if scale is None:
        scale = jnp.ones((Cin,), jnp.float32)
        shift = jnp.zeros((Cin,), jnp.float32)
    kern = functools.partial(_pconv_kernel, BI=bi, h=h, w=w, up=up,
                             pre_act=pre_act, slope=slope, act=act,
                             stats=stats, taps=taps, P_in=P_in)
    out_shape = [jax.ShapeDtypeStruct((B, P_out, H, w, Cout), out_dtype)]
    out_specs = [pl.BlockSpec((bi, P_out, H, w, Cout),
                              lambda i: (i, 0, 0, 0, 0))]
    if stats:
        out_shape += [jax.ShapeDtypeStruct((B // bi, 1, Cout), jnp.float32)] * 2
        out_specs += [pl.BlockSpec((1, 1, Cout), lambda i: (i, 0, 0))] * 2
    res = pl.pallas_call(
        kern,
        out_shape=tuple(out_shape),
        grid_spec=pltpu.PrefetchScalarGridSpec(
            num_scalar_prefetch=0,
            grid=(B // bi,),
            in_specs=[
                pl.BlockSpec((bi, P_in, h, w, Cin), lambda i: (i, 0, 0, 0, 0)),
                pl.BlockSpec((1, Cin), lambda i: (0, 0)),
                pl.BlockSpec((1, Cin), lambda i: (0, 0)),
                pl.BlockSpec((T, Cin, Cout), lambda i: (0, 0, 0)),
                pl.BlockSpec((1, Cout), lambda i: (0, 0)),
            ],
            out_specs=out_specs,
            scratch_shapes=[
                pltpu.VMEM((P_in, h + 2, w + 2 * _PAD, Cin), jnp.bfloat16)],
        ),
        compiler_params=pltpu.CompilerParams(
            dimension_semantics=("parallel",),
            vmem_limit_bytes=100 * 1024 * 1024,
        ),
    )(x, scale.reshape(1, Cin), shift.reshape(1, Cin),
      wt.astype(jnp.bfloat16), b.reshape(1, Cout))
    return res if stats else res[0]


def _comb_weights(w):
    m = jnp.array([[1, 0, 0], [0, 1, 1], [1, 1, 0], [0, 0, 1]], w.dtype)
    wc = jnp.einsum('id,jw,dwco->ijco', m, m, w)
    return wc.reshape(16, w.shape[2], w.shape[3])


_T1 = [
    [(0, -1, 0), (0, 0, 1)],
    [(0, 0, 2), (0, 1, 3)],
]
_T2 = [
    [(1, -1, 0), (0, 0, 1)],
    [(0, 0, 2), (1, 0, 3)],
    [(0, 0, 0), (1, 0, 1)],
    [(1, 0, 2), (0, 1, 3)],
]
_T3 = [
    [((p + dw - 1) % 4, dh, (p + dw - 1) // 4, 3 * dh + dw)
     for dw in range(3) for dh in range(3)]
    for p in range(4)
]


def _bn_scale_shift(ssum, ssq, gamma, beta, count, eps=0.8):
    mean = jnp.sum(ssum, axis=(0, 1)) / count
    var = jnp.sum(ssq, axis=(0, 1)) / count - mean * mean
    scale = gamma * jax.lax.rsqrt(var + eps)
    shift = beta - mean * scale
    return scale, shift


def kernel(z, l1_w, l1_b, c1_w, c1_b, bn1_g, bn1_b,
           c2_w, c2_b, bn2_g, bn2_b, c3_w, c3_b):
    B = z.shape[0]
    init = 16

    h = _linear(z, l1_w, l1_b)
    x = h.reshape(B, 1, init, init, 128)

    c1, s1, q1 = _pconv(x, _comb_weights(c1_w), c1_b, taps=_T1, P_out=2,
                        up=True)
    sc1, sh1 = _bn_scale_shift(s1, q1, bn1_g, bn1_b, B * 32 * 32)

    c2, s2, q2 = _pconv(c1, _comb_weights(c2_w), c2_b, taps=_T2, P_out=4,
                        up=True, scale=sc1, shift=sh1)
    sc2, sh2 = _bn_scale_shift(s2, q2, bn2_g, bn2_b, B * 64 * 64)

    c3 = _pconv(c2, c3_w.reshape(9, 64, 3), c3_b, taps=_T3, P_out=4,
                up=False, scale=sc2, shift=sh2, act="tanh",
                out_dtype=jnp.float32, stats=False)

    out = c3.transpose(0, 4, 2, 3, 1).reshape(B, 3, 64, 64)
    return out


# speedup vs baseline: 1.0089x; 1.0058x over previous
import functools

import jax
import jax.numpy as jnp
from jax.experimental import pallas as pl
from jax.experimental.pallas import tpu as pltpu

_PAD = 8


def _linear_kernel(z_ref, w_ref, b_ref, o_ref):
    zb = z_ref[...].astype(jnp.bfloat16)
    wb = w_ref[...].astype(jnp.bfloat16)
    acc = jnp.dot(zb, wb, preferred_element_type=jnp.float32)
    o_ref[...] = (acc + b_ref[...]).astype(o_ref.dtype)


def _linear(z, w, b, *, tn=8192):
    B, K = z.shape
    N = w.shape[1]
    tn = min(tn, N)
    return pl.pallas_call(
        _linear_kernel,
        out_shape=jax.ShapeDtypeStruct((B, N), jnp.bfloat16),
        grid=(N // tn,),
        in_specs=[
            pl.BlockSpec((B, K), lambda j: (0, 0)),
            pl.BlockSpec((K, tn), lambda j: (0, j)),
            pl.BlockSpec((1, tn), lambda j: (0, j)),
        ],
        out_specs=pl.BlockSpec((B, tn), lambda j: (0, j)),
        compiler_params=pltpu.CompilerParams(dimension_semantics=("parallel",)),
    )(z, w, b.reshape(1, N))


def _pconv_kernel(x_ref, s_ref, t_ref, w_ref, b_ref, *rest,
                  BI, h, w, up, pre_act, slope, act, stats, taps, P_in):
    if stats:
        o_ref, sum_ref, sq_ref, pad_ref = rest
    else:
        o_ref, pad_ref = rest
    pad_ref[...] = jnp.zeros_like(pad_ref)
    tsum = tsq = None
    for bi in range(BI):
        for pin in range(P_in):
            v = x_ref[bi, pin].astype(jnp.float32)
            if pre_act:
                v = v * s_ref[...] + t_ref[...]
                v = jnp.where(v >= 0.0, v, slope * v)
            pad_ref[pin, 1:h + 1, _PAD:_PAD + w, :] = v.astype(jnp.bfloat16)

        for k, tap_list in enumerate(taps):
            if up:
                ys = []
                for a in range(2):
                    acc = None
                    for (pin, dc, cc) in tap_list:
                        for u in range(2):
                            patch = pad_ref[pin, a + u:a + u + h,
                                            _PAD + dc:_PAD + dc + w, :]
                            d = jax.lax.dot_general(
                                patch, w_ref[(2 * a + u) * 4 + cc],
                                dimension_numbers=(((2,), (0,)), ((), ())),
                                preferred_element_type=jnp.float32,
                            )
                            acc = d if acc is None else acc + d
                    ys.append(acc + b_ref[...])
                cout = ys[0].shape[-1]
                y = jnp.stack(ys, axis=1).reshape(2 * h, w, cout)
            else:
                acc = None
                for (pin, dh, dc, t) in tap_list:
                    patch = pad_ref[pin, dh:dh + h,
                                    _PAD + dc:_PAD + dc + w, :]
                    d = jax.lax.dot_general(
                        patch, w_ref[t],
                        dimension_numbers=(((2,), (0,)), ((), ())),
                        preferred_element_type=jnp.float32,
                    )
                    acc = d if acc is None else acc + d
                y = acc + b_ref[...]
            if act == "tanh":
                y = jnp.tanh(y)
            o_ref[bi, k] = y.astype(o_ref.dtype)
            if stats:
                s = jnp.sum(jnp.sum(y, axis=0), axis=0, keepdims=True)
                q = jnp.sum(jnp.sum(y * y, axis=0), axis=0, keepdims=True)
                tsum = s if tsum is None else tsum + s
                tsq = q if tsq is None else tsq + q
    if stats:
        sum_ref[0] = tsum
        sq_ref[0] = tsq


def _pconv(x, wt, b, *, taps, P_out, up, scale=None, shift=None, slope=0.2,
           act="none", out_dtype=jnp.bfloat16, stats=True, bi=8):
    B, P_in, h, w, Cin = x.shape
    H = 2 * h if up else h
    T = wt.shape[0]
    Cout = wt.shape[-1]
    bi = max(1, min(bi, B))
    pre_act = scale is not None
    if scale is None:
        scale = jnp.ones((Cin,), jnp.float32)
        shift = jnp.zeros((Cin,), jnp.float32)
    kern = functools.partial(_pconv_kernel, BI=bi, h=h, w=w, up=up,
                             pre_act=pre_act, slope=slope, act=act,
                             stats=stats, taps=taps, P_in=P_in)
    out_shape = [jax.ShapeDtypeStruct((B, P_out, H, w, Cout), out_dtype)]
    out_specs = [pl.BlockSpec((bi, P_out, H, w, Cout),
                              lambda i: (i, 0, 0, 0, 0))]
    if stats:
        out_shape += [jax.ShapeDtypeStruct((B // bi, 1, Cout), jnp.float32)] * 2
        out_specs += [pl.BlockSpec((1, 1, Cout), lambda i: (i, 0, 0))] * 2
    res = pl.pallas_call(
        kern,
        out_shape=tuple(out_shape),
        grid_spec=pltpu.PrefetchScalarGridSpec(
            num_scalar_prefetch=0,
            grid=(B // bi,),
            in_specs=[
                pl.BlockSpec((bi, P_in, h, w, Cin), lambda i: (i, 0, 0, 0, 0)),
                pl.BlockSpec((1, Cin), lambda i: (0, 0)),
                pl.BlockSpec((1, Cin), lambda i: (0, 0)),
                pl.BlockSpec((T, Cin, Cout), lambda i: (0, 0, 0)),
                pl.BlockSpec((1, Cout), lambda i: (0, 0)),
            ],
            out_specs=out_specs,
            scratch_shapes=[
                pltpu.VMEM((P_in, h + 2, w + 2 * _PAD, Cin), jnp.bfloat16)],
        ),
        compiler_params=pltpu.CompilerParams(
            dimension_semantics=("parallel",),
            vmem_limit_bytes=100 * 1024 * 1024,
        ),
    )(x, scale.reshape(1, Cin), shift.reshape(1, Cin),
      wt.astype(jnp.bfloat16), b.reshape(1, Cout))
    return res if stats else res[0]


def _comb_weights(w):
    m = jnp.array([[1, 0, 0], [0, 1, 1], [1, 1, 0], [0, 0, 1]], w.dtype)
    wc = jnp.einsum('id,jw,dwco->ijco', m, m, w)
    return wc.reshape(16, w.shape[2], w.shape[3])


_T1 = [
    [(0, -1, 0), (0, 0, 1)],
    [(0, 0, 2), (0, 1, 3)],
]
_T2 = [
    [(1, -1, 0), (0, 0, 1)],
    [(0, 0, 2), (1, 0, 3)],
    [(0, 0, 0), (1, 0, 1)],
    [(1, 0, 2), (0, 1, 3)],
]
_T3 = [
    [((p + dw - 1) % 4, dh, (p + dw - 1) // 4, 3 * dh + dw)
     for dw in range(3) for dh in range(3)]
    for p in range(4)
]


def _bn_scale_shift(ssum, ssq, gamma, beta, count, eps=0.8):
    mean = jnp.sum(ssum, axis=(0, 1)) / count
    var = jnp.sum(ssq, axis=(0, 1)) / count - mean * mean
    scale = gamma * jax.lax.rsqrt(var + eps)
    shift = beta - mean * scale
    return scale, shift


def kernel(z, l1_w, l1_b, c1_w, c1_b, bn1_g, bn1_b,
           c2_w, c2_b, bn2_g, bn2_b, c3_w, c3_b):
    B = z.shape[0]
    init = 16

    h = _linear(z, l1_w, l1_b)
    x = h.reshape(B, 1, init, init, 128)

    c1, s1, q1 = _pconv(x, _comb_weights(c1_w), c1_b, taps=_T1, P_out=2,
                        up=True)
    sc1, sh1 = _bn_scale_shift(s1, q1, bn1_g, bn1_b, B * 32 * 32)

    c2, s2, q2 = _pconv(c1, _comb_weights(c2_w), c2_b, taps=_T2, P_out=4,
                        up=True, scale=sc1, shift=sh1)
    sc2, sh2 = _bn_scale_shift(s2, q2, bn2_g, bn2_b, B * 64 * 64)

    c3 = _pconv(c2, c3_w.reshape(9, 64, 3), c3_b, taps=_T3, P_out=4,
                up=False, scale=sc2, shift=sh2, act="tanh",
                out_dtype=jnp.float32, stats=False)

    out = c3.transpose(0, 4, 2, 3, 1).reshape(B, 3, 64, 64)
    return out


# speedup vs baseline: 1.0402x; 1.0310x over previous
import functools

import jax
import jax.numpy as jnp
from jax.experimental import pallas as pl
from jax.experimental.pallas import tpu as pltpu

_PAD = 8


def _linear_kernel(z_ref, w_ref, b_ref, o_ref):
    zb = z_ref[...].astype(jnp.bfloat16)
    wb = w_ref[...].astype(jnp.bfloat16)
    acc = jnp.dot(zb, wb, preferred_element_type=jnp.float32)
    o_ref[...] = (acc + b_ref[...]).astype(o_ref.dtype)


def _linear(z, w, b, *, tn=8192):
    B, K = z.shape
    N = w.shape[1]
    tn = min(tn, N)
    return pl.pallas_call(
        _linear_kernel,
        out_shape=jax.ShapeDtypeStruct((B, N), jnp.bfloat16),
        grid=(N // tn,),
        in_specs=[
            pl.BlockSpec((B, K), lambda j: (0, 0)),
            pl.BlockSpec((K, tn), lambda j: (0, j)),
            pl.BlockSpec((1, tn), lambda j: (0, j)),
        ],
        out_specs=pl.BlockSpec((B, tn), lambda j: (0, j)),
        compiler_params=pltpu.CompilerParams(dimension_semantics=("parallel",)),
    )(z, w, b.reshape(1, N))


def _pconv_kernel(x_ref, s_ref, t_ref, w_ref, b_ref, *rest,
                  BI, h, w, up, pre_act, slope, act, stats, taps, P_in):
    if stats:
        o_ref, sum_ref, sq_ref, pad_ref = rest
    else:
        o_ref, pad_ref = rest
    pad_ref[...] = jnp.zeros_like(pad_ref)
    tsum = tsq = None
    for bi in range(BI):
        for pin in range(P_in):
            v = x_ref[bi, pin]
            if pre_act:
                v = v * s_ref[...] + t_ref[...]
                v = jnp.where(v >= 0, v, jnp.bfloat16(slope) * v)
            pad_ref[pin, 1:h + 1, _PAD:_PAD + w, :] = v

        for k, tap_list in enumerate(taps):
            if up:
                ys = []
                for a in range(2):
                    acc = None
                    for (pin, dc, cc) in tap_list:
                        for u in range(2):
                            patch = pad_ref[pin, a + u:a + u + h,
                                            _PAD + dc:_PAD + dc + w, :]
                            d = jax.lax.dot_general(
                                patch, w_ref[(2 * a + u) * 4 + cc],
                                dimension_numbers=(((2,), (0,)), ((), ())),
                                preferred_element_type=jnp.float32,
                            )
                            acc = d if acc is None else acc + d
                    ys.append(acc + b_ref[...])
                cout = ys[0].shape[-1]
                y = jnp.stack(ys, axis=1).reshape(2 * h, w, cout)
            else:
                acc = None
                for (pin, dh, dc, t) in tap_list:
                    patch = pad_ref[pin, dh:dh + h,
                                    _PAD + dc:_PAD + dc + w, :]
                    d = jax.lax.dot_general(
                        patch, w_ref[t],
                        dimension_numbers=(((2,), (0,)), ((), ())),
                        preferred_element_type=jnp.float32,
                    )
                    acc = d if acc is None else acc + d
                y = acc + b_ref[...]
            if act == "tanh":
                y = jnp.tanh(y)
            o_ref[bi, k] = y.astype(o_ref.dtype)
            if stats:
                s = jnp.sum(jnp.sum(y, axis=0), axis=0, keepdims=True)
                q = jnp.sum(jnp.sum(y * y, axis=0), axis=0, keepdims=True)
                tsum = s if tsum is None else tsum + s
                tsq = q if tsq is None else tsq + q
    if stats:
        sum_ref[0] = tsum
        sq_ref[0] = tsq


def _pconv(x, wt, b, *, taps, P_out, up, scale=None, shift=None, slope=0.2,
           act="none", out_dtype=jnp.bfloat16, stats=True, bi=8):
    B, P_in, h, w, Cin = x.shape
    H = 2 * h if up else h
    T = wt.shape[0]
    Cout = wt.shape[-1]
    bi = max(1, min(bi, B))
    pre_act = scale is not None
    if scale is None:
        scale = jnp.ones((Cin,), jnp.float32)
        shift = jnp.zeros((Cin,), jnp.float32)
    kern = functools.partial(_pconv_kernel, BI=bi, h=h, w=w, up=up,
                             pre_act=pre_act, slope=slope, act=act,
                             stats=stats, taps=taps, P_in=P_in)
    out_shape = [jax.ShapeDtypeStruct((B, P_out, H, w, Cout), out_dtype)]
    out_specs = [pl.BlockSpec((bi, P_out, H, w, Cout),
                              lambda i: (i, 0, 0, 0, 0))]
    if stats:
        out_shape += [jax.ShapeDtypeStruct((B // bi, 1, Cout), jnp.float32)] * 2
        out_specs += [pl.BlockSpec((1, 1, Cout), lambda i: (i, 0, 0))] * 2
    res = pl.pallas_call(
        kern,
        out_shape=tuple(out_shape),
        grid_spec=pltpu.PrefetchScalarGridSpec(
            num_scalar_prefetch=0,
            grid=(B // bi,),
            in_specs=[
                pl.BlockSpec((bi, P_in, h, w, Cin), lambda i: (i, 0, 0, 0, 0)),
                pl.BlockSpec((1, Cin), lambda i: (0, 0)),
                pl.BlockSpec((1, Cin), lambda i: (0, 0)),
                pl.BlockSpec((T, Cin, Cout), lambda i: (0, 0, 0)),
                pl.BlockSpec((1, Cout), lambda i: (0, 0)),
            ],
            out_specs=out_specs,
            scratch_shapes=[
                pltpu.VMEM((P_in, h + 2, w + 2 * _PAD, Cin), jnp.bfloat16)],
        ),
        compiler_params=pltpu.CompilerParams(
            dimension_semantics=("parallel",),
            vmem_limit_bytes=100 * 1024 * 1024,
        ),
    )(x, scale.reshape(1, Cin).astype(jnp.bfloat16),
      shift.reshape(1, Cin).astype(jnp.bfloat16),
      wt.astype(jnp.bfloat16), b.reshape(1, Cout))
    return res if stats else res[0]


def _comb_weights(w):
    m = jnp.array([[1, 0, 0], [0, 1, 1], [1, 1, 0], [0, 0, 1]], w.dtype)
    wc = jnp.einsum('id,jw,dwco->ijco', m, m, w)
    return wc.reshape(16, w.shape[2], w.shape[3])


_T1 = [
    [(0, -1, 0), (0, 0, 1)],
    [(0, 0, 2), (0, 1, 3)],
]
_T2 = [
    [(1, -1, 0), (0, 0, 1)],
    [(0, 0, 2), (1, 0, 3)],
    [(0, 0, 0), (1, 0, 1)],
    [(1, 0, 2), (0, 1, 3)],
]
_T3 = [
    [((p + dw - 1) % 4, dh, (p + dw - 1) // 4, 3 * dh + dw)
     for dw in range(3) for dh in range(3)]
    for p in range(4)
]


def _bn_scale_shift(ssum, ssq, gamma, beta, count, eps=0.8):
    mean = jnp.sum(ssum, axis=(0, 1)) / count
    var = jnp.sum(ssq, axis=(0, 1)) / count - mean * mean
    scale = gamma * jax.lax.rsqrt(var + eps)
    shift = beta - mean * scale
    return scale, shift


def kernel(z, l1_w, l1_b, c1_w, c1_b, bn1_g, bn1_b,
           c2_w, c2_b, bn2_g, bn2_b, c3_w, c3_b):
    B = z.shape[0]
    init = 16

    h = _linear(z, l1_w, l1_b)
    x = h.reshape(B, 1, init, init, 128)

    c1, s1, q1 = _pconv(x, _comb_weights(c1_w), c1_b, taps=_T1, P_out=2,
                        up=True)
    sc1, sh1 = _bn_scale_shift(s1, q1, bn1_g, bn1_b, B * 32 * 32)

    c2, s2, q2 = _pconv(c1, _comb_weights(c2_w), c2_b, taps=_T2, P_out=4,
                        up=True, scale=sc1, shift=sh1)
    sc2, sh2 = _bn_scale_shift(s2, q2, bn2_g, bn2_b, B * 64 * 64)

    c3 = _pconv(c2, c3_w.reshape(9, 64, 3), c3_b, taps=_T3, P_out=4,
                up=False, scale=sc2, shift=sh2, act="tanh",
                out_dtype=jnp.float32, stats=False)

    out = c3.transpose(0, 4, 2, 3, 1).reshape(B, 3, 64, 64)
    return out
